# baseline (speedup 1.0000x reference)
import functools

import jax
import jax.numpy as jnp
from jax import lax
from jax.experimental import pallas as pl
from jax.experimental.pallas import tpu as pltpu
from jax.experimental.pallas import tpu_sc as plsc

B = 16384
E = 128
NC = 2
NS = 16
L = 16
NW = NC * NS
BPW = B // NW
C = 128
NCHUNK = BPW // C
NROW = B // C


_mesh = plsc.VectorSubcoreMesh(core_axis_name="c", subcore_axis_name="s")


@functools.partial(
    pl.kernel,
    mesh=_mesh,
    out_type=jax.ShapeDtypeStruct((NW * L,), jnp.float32),
    scratch_types=[
        pltpu.VMEM((BPW,), jnp.int32),
        pltpu.VMEM((BPW,), jnp.int32),
        pltpu.VMEM((2, C, E), jnp.float32),
        pltpu.VMEM((2, C, E), jnp.float32),
        pltpu.VMEM((L,), jnp.float32),
        pltpu.SemaphoreType.DMA,
        pltpu.SemaphoreType.DMA,
        pltpu.SemaphoreType.DMA,
        pltpu.SemaphoreType.DMA,
    ],
)
def _sc_gather_dot(u_idx_hbm, m_idx_hbm, uemb_hbm, memb_hbm, part_hbm,
                   uidx_v, midx_v, urow_v, mrow_v, acc_v,
                   sem_u0, sem_u1, sem_m0, sem_m1):
    wid = lax.axis_index("s") * NC + lax.axis_index("c")
    base = wid * BPW
    pltpu.sync_copy(u_idx_hbm.at[pl.ds(base, BPW)], uidx_v)
    pltpu.sync_copy(m_idx_hbm.at[pl.ds(base, BPW)], midx_v)
    sem_u = (sem_u0, sem_u1)
    sem_m = (sem_m0, sem_m1)

    def fire(c):
        b = c & 1
        return (pltpu.async_copy(uemb_hbm.at[uidx_v.at[pl.ds(c * C, C)]],
                                 urow_v.at[b], sem_u[b]),
                pltpu.async_copy(memb_hbm.at[midx_v.at[pl.ds(c * C, C)]],
                                 mrow_v.at[b], sem_m[b]))

    acc = jnp.zeros((L,), jnp.float32)
    cps = [None, None]
    cps[0] = fire(0)
    for c in range(NCHUNK):
        b = c & 1
        if c + 1 < NCHUNK:
            cps[1 - b] = fire(c + 1)
        cps[b][0].wait()
        cps[b][1].wait()

        def body(r, a, b=b):
            for e in range(E // L):
                a = a + (urow_v[b, r, pl.ds(e * L, L)]
                         * mrow_v[b, r, pl.ds(e * L, L)])
            return a

        acc = lax.fori_loop(0, C, body, acc, unroll=2)
    acc_v[...] = acc
    pltpu.sync_copy(acc_v, part_hbm.at[pl.ds(wid * L, L)])


@functools.partial(
    pl.kernel,
    mesh=_mesh,
    out_type=jax.ShapeDtypeStruct((B,), jnp.float32),
    scratch_types=[
        pltpu.VMEM((BPW,), jnp.int32),
        pltpu.VMEM((BPW,), jnp.int32),
        pltpu.VMEM((BPW,), jnp.float32),
        pltpu.VMEM((BPW,), jnp.float32),
        pltpu.VMEM((NW * L,), jnp.float32),
        pltpu.VMEM((BPW,), jnp.float32),
        pltpu.SemaphoreType.DMA,
        pltpu.SemaphoreType.DMA,
        pltpu.SemaphoreType.DMA,
    ],
)
def _sc_combine(u_idx_hbm, m_idx_hbm, ubias_hbm, mbias_hbm, part_hbm,
                out_hbm, uidx_v, midx_v, ub_v, mb_v, part_v, out_v,
                sem_p, sem_ub, sem_mb):
    wid = lax.axis_index("s") * NC + lax.axis_index("c")
    base = wid * BPW
    p_cp = pltpu.async_copy(part_hbm, part_v, sem_p)
    pltpu.sync_copy(u_idx_hbm.at[pl.ds(base, BPW)], uidx_v)
    pltpu.sync_copy(m_idx_hbm.at[pl.ds(base, BPW)], midx_v)
    bias_cps = []
    for c in range(NCHUNK):
        sl = pl.ds(c * C, C)
        bias_cps.append(
            pltpu.async_copy(ubias_hbm.at[uidx_v.at[sl]], ub_v.at[sl],
                             sem_ub))
        bias_cps.append(
            pltpu.async_copy(mbias_hbm.at[midx_v.at[sl]], mb_v.at[sl],
                             sem_mb))
    p_cp.wait()
    s16 = jnp.zeros((L,), jnp.float32)
    for j in range(NW):
        s16 = s16 + part_v[pl.ds(j * L, L)]
    s = s16[0]
    for i in range(1, L):
        s = s + s16[i]
    for cp in bias_cps:
        cp.wait()
    for j in range(BPW // L):
        x = s + ub_v[pl.ds(j * L, L)] + mb_v[pl.ds(j * L, L)]
        out_v[pl.ds(j * L, L)] = 1.0 / (1.0 + jnp.exp(-x))
    pltpu.sync_copy(out_v, out_hbm.at[pl.ds(base, BPW)])


def kernel(inputs, user_emb, user_bias_table, movie_emb, movie_bias_table):
    u_idx = inputs[:, 0]
    m_idx = inputs[:, 1]
    ub_flat = user_bias_table.reshape(-1)
    mb_flat = movie_bias_table.reshape(-1)
    partials = _sc_gather_dot(u_idx, m_idx, user_emb, movie_emb)
    out = _sc_combine(u_idx, m_idx, ub_flat, mb_flat, partials)
    return out.reshape(B, 1)


# speedup vs baseline: 1.0866x; 1.0866x over previous
import functools

import jax
import jax.numpy as jnp
from jax import lax
from jax.experimental import pallas as pl
from jax.experimental.pallas import tpu as pltpu
from jax.experimental.pallas import tpu_sc as plsc

B = 16384
E = 128
NC = 2
NS = 16
L = 16
NW = NC * NS
BPW = B // NW
C = 128
NCHUNK = BPW // C
NROW = B // C


_mesh = plsc.VectorSubcoreMesh(core_axis_name="c", subcore_axis_name="s")


@functools.partial(
    pl.kernel,
    mesh=_mesh,
    out_type=[
        jax.ShapeDtypeStruct((NW * L,), jnp.float32),
        jax.ShapeDtypeStruct((NROW, C), jnp.float32),
        jax.ShapeDtypeStruct((NROW, C), jnp.float32),
    ],
    scratch_types=[
        pltpu.VMEM((NCHUNK, C), jnp.int32),
        pltpu.VMEM((NCHUNK, C), jnp.int32),
        pltpu.VMEM((2, C, E), jnp.float32),
        pltpu.VMEM((2, C, E), jnp.float32),
        pltpu.VMEM((NCHUNK, C), jnp.float32),
        pltpu.VMEM((NCHUNK, C), jnp.float32),
        pltpu.VMEM((L,), jnp.float32),
        pltpu.SemaphoreType.DMA,
        pltpu.SemaphoreType.DMA,
        pltpu.SemaphoreType.DMA,
        pltpu.SemaphoreType.DMA,
        pltpu.SemaphoreType.DMA,
        pltpu.SemaphoreType.DMA,
    ],
)
def _sc_gather_dot(u_idx_hbm, m_idx_hbm, uemb_hbm, ubias_hbm, memb_hbm,
                   mbias_hbm, part_hbm, ubg_hbm, mbg_hbm,
                   uidx_v, midx_v, urow_v, mrow_v, ub_v, mb_v, acc_v,
                   sem_u0, sem_u1, sem_m0, sem_m1, sem_ub, sem_mb):
    wid = lax.axis_index("s") * NC + lax.axis_index("c")
    cbase = wid * NCHUNK
    pltpu.sync_copy(u_idx_hbm.at[pl.ds(cbase, NCHUNK)], uidx_v)
    pltpu.sync_copy(m_idx_hbm.at[pl.ds(cbase, NCHUNK)], midx_v)
    sem_u = (sem_u0, sem_u1)
    sem_m = (sem_m0, sem_m1)

    def fire(c):
        b = c & 1
        return (pltpu.async_copy(uemb_hbm.at[uidx_v.at[c]], urow_v.at[b],
                                 sem_u[b]),
                pltpu.async_copy(memb_hbm.at[midx_v.at[c]], mrow_v.at[b],
                                 sem_m[b]))

    acc = jnp.zeros((L,), jnp.float32)
    cps = [None, None]
    bias_cps = []
    cps[0] = fire(0)
    for c in range(NCHUNK):
        b = c & 1
        if c + 1 < NCHUNK:
            cps[1 - b] = fire(c + 1)
        bias_cps.append(
            pltpu.async_copy(ubias_hbm.at[uidx_v.at[c]], ub_v.at[c], sem_ub))
        bias_cps.append(
            pltpu.async_copy(mbias_hbm.at[midx_v.at[c]], mb_v.at[c], sem_mb))
        cps[b][0].wait()
        cps[b][1].wait()

        def body(r, a, b=b):
            for e in range(E // L):
                a = a + (urow_v[b, r, pl.ds(e * L, L)]
                         * mrow_v[b, r, pl.ds(e * L, L)])
            return a

        acc = lax.fori_loop(0, C, body, acc, unroll=2)
    acc_v[...] = acc
    pltpu.sync_copy(acc_v, part_hbm.at[pl.ds(wid * L, L)])
    for cp in bias_cps:
        cp.wait()
    pltpu.sync_copy(ub_v, ubg_hbm.at[pl.ds(cbase, NCHUNK)])
    pltpu.sync_copy(mb_v, mbg_hbm.at[pl.ds(cbase, NCHUNK)])


def _combine(p_ref, ub_ref, mb_ref, o_ref):
    s = jnp.sum(p_ref[...])
    o_ref[...] = jax.nn.sigmoid(s + ub_ref[...] + mb_ref[...])


def kernel(inputs, user_emb, user_bias_table, movie_emb, movie_bias_table):
    u_idx = inputs[:, 0].reshape(NROW, C)
    m_idx = inputs[:, 1].reshape(NROW, C)
    ub_flat = jnp.pad(user_bias_table, ((0, 352), (0, 0))).reshape(-1)
    mb_flat = jnp.pad(movie_bias_table, ((0, 352), (0, 0))).reshape(-1)
    partials, ubg, mbg = _sc_gather_dot(
        u_idx, m_idx, user_emb, ub_flat, movie_emb, mb_flat)
    out2d = pl.pallas_call(
        _combine,
        out_shape=jax.ShapeDtypeStruct((NROW, C), jnp.float32),
    )(partials.reshape(NW * L // C, C), ubg, mbg)
    return out2d.reshape(B, 1)
